# Initial kernel scaffold; baseline (speedup 1.0000x reference)
import jax
import jax.numpy as jnp
from jax.experimental import pallas as pl


def kernel(z, embedding):
    raise NotImplementedError("write your pallas kernel here")


# speedup vs baseline: 3.0980x; 3.0980x over previous
import functools

import jax
import jax.numpy as jnp
from jax.experimental import pallas as pl
from jax.experimental.pallas import tpu as pltpu

NUM_EMBEDDINGS = 1024
EMBEDDING_DIM = 64
BETA = 0.25
B = 16
P = 1024
N_TOTAL = B * P


def _vq_kernel(zb_ref, emb_ref, z2_ref, e2_ref,
               loss_ref, zq_ref, perp_ref, enc_ref, idx_ref,
               counts_ref, sse_ref):
    b = pl.program_id(0)

    zb = zb_ref[0]
    emb = emb_ref[...]
    z2 = z2_ref[0]
    e2 = e2_ref[...]

    m = jax.lax.dot_general(
        zb, emb, dimension_numbers=(((0,), (1,)), ((), ())),
        preferred_element_type=jnp.float32)
    d = (z2 + e2) - 2.0 * m

    minv = jnp.min(d, axis=1, keepdims=True)
    iota_k = jax.lax.broadcasted_iota(jnp.int32, (P, NUM_EMBEDDINGS), 1)
    idx = jnp.min(jnp.where(d == minv, iota_k, NUM_EMBEDDINGS),
                  axis=1, keepdims=True)
    one_hot = (iota_k == idx).astype(jnp.float32)

    enc_ref[...] = one_hot
    idx_ref[...] = idx

    zq_t = jax.lax.dot_general(
        emb, one_hot, dimension_numbers=(((0,), (1,)), ((), ())),
        preferred_element_type=jnp.float32)
    zq_st = zb + (zq_t - zb)
    zq_ref[0] = zq_st

    diff = zq_t - zb
    part_sse = jnp.sum(diff * diff)
    part_counts = jnp.sum(one_hot, axis=0, keepdims=True)

    @pl.when(b == 0)
    def _init():
        sse_ref[0, 0] = part_sse
        counts_ref[...] = part_counts

    @pl.when(b > 0)
    def _acc():
        sse_ref[0, 0] += part_sse
        counts_ref[...] += part_counts

    @pl.when(b == B - 1)
    def _finalize():
        sse = sse_ref[0, 0]
        loss_ref[...] = jnp.reshape(
            (1.0 + BETA) * sse / float(N_TOTAL * EMBEDDING_DIM), (1, 1))
        me = counts_ref[...] / float(N_TOTAL)
        perp_ref[...] = jnp.reshape(
            jnp.exp(-jnp.sum(me + jnp.log(me + 1e-10))), (1, 1))


@functools.partial(jax.jit, static_argnames=())
def kernel(z, embedding):
    zp = jnp.transpose(z, (0, 2, 3, 1))
    z_flat = zp.reshape(-1, EMBEDDING_DIM)
    z2 = jnp.sum(z_flat ** 2, axis=1)
    e2 = jnp.sum(embedding ** 2, axis=1)

    z_cp = z.reshape(B, EMBEDDING_DIM, P)
    z2_r = z2.reshape(B, P, 1)
    e2_r = e2.reshape(1, NUM_EMBEDDINGS)

    grid = (B,)
    out_shapes = (
        jax.ShapeDtypeStruct((1, 1), jnp.float32),
        jax.ShapeDtypeStruct((B, EMBEDDING_DIM, P), jnp.float32),
        jax.ShapeDtypeStruct((1, 1), jnp.float32),
        jax.ShapeDtypeStruct((N_TOTAL, NUM_EMBEDDINGS), jnp.float32),
        jax.ShapeDtypeStruct((N_TOTAL, 1), jnp.int32),
    )
    in_specs = [
        pl.BlockSpec((1, EMBEDDING_DIM, P), lambda b: (b, 0, 0)),
        pl.BlockSpec((NUM_EMBEDDINGS, EMBEDDING_DIM), lambda b: (0, 0)),
        pl.BlockSpec((1, P, 1), lambda b: (b, 0, 0)),
        pl.BlockSpec((1, NUM_EMBEDDINGS), lambda b: (0, 0)),
    ]
    out_specs = (
        pl.BlockSpec((1, 1), lambda b: (0, 0)),
        pl.BlockSpec((1, EMBEDDING_DIM, P), lambda b: (b, 0, 0)),
        pl.BlockSpec((1, 1), lambda b: (0, 0)),
        pl.BlockSpec((P, NUM_EMBEDDINGS), lambda b: (b, 0)),
        pl.BlockSpec((P, 1), lambda b: (b, 0)),
    )
    loss, zq, perp, enc, idx = pl.pallas_call(
        _vq_kernel,
        grid=grid,
        in_specs=in_specs,
        out_specs=out_specs,
        out_shape=out_shapes,
        scratch_shapes=[
            pltpu.VMEM((1, NUM_EMBEDDINGS), jnp.float32),
            pltpu.SMEM((1, 1), jnp.float32),
        ],
    )(z_cp, embedding, z2_r, e2_r)

    z_q_out = zq.reshape(z.shape)
    return (loss[0, 0], z_q_out, perp[0, 0], enc, idx)
